# Initial kernel scaffold; baseline (speedup 1.0000x reference)
import jax
import jax.numpy as jnp
from jax.experimental import pallas as pl


def kernel(thresholds, rules_left, rules_right, binary_ops, left, right, leaf_is_reroll, leaf_mask_left, leaf_mask_right, leaf_mask_op, leaf_score_cat):
    raise NotImplementedError("write your pallas kernel here")


# speedup vs baseline: 11282.3368x; 11282.3368x over previous
import jax
import jax.numpy as jnp
from jax import lax
from jax.experimental import pallas as pl
from jax.experimental.pallas import tpu as pltpu
from jax.experimental.pallas import tpu_sc as plsc

_N = 10000
_LANES = 16
_CHUNKS = _N // _LANES


def _reach_body(left_hbm, right_hbm, out_hbm, left_v, right_v, reach_v):
    c = lax.axis_index("c")
    s = lax.axis_index("s")

    @pl.when((c == 0) & (s == 0))
    def _():
        pltpu.sync_copy(left_hbm, left_v)
        pltpu.sync_copy(right_hbm, right_v)

        zeros = jnp.zeros((_LANES,), jnp.int32)
        ones = jnp.ones((_LANES,), jnp.int32)

        def init_chunk(i, carry):
            reach_v[pl.ds(i * _LANES, _LANES)] = zeros
            return carry

        lax.fori_loop(0, _CHUNKS, init_chunk, 0)
        lane = lax.iota(jnp.int32, _LANES)
        reach_v[pl.ds(0, _LANES)] = jnp.where(lane == zeros, ones, zeros)

        def sweep_chunk(i, carry):
            r = reach_v[pl.ds(i * _LANES, _LANES)]
            m = r != zeros
            l = left_v[pl.ds(i * _LANES, _LANES)]
            rr = right_v[pl.ds(i * _LANES, _LANES)]
            plsc.store_scatter(reach_v, [l], ones, mask=m & (l >= zeros))
            plsc.store_scatter(reach_v, [rr], ones, mask=m & (rr >= zeros))
            return carry

        def count_chunk(i, acc):
            return acc + reach_v[pl.ds(i * _LANES, _LANES)]

        def cond(st):
            return st[0] != st[1]

        def wbody(st):
            _, cur = st
            lax.fori_loop(0, _CHUNKS, sweep_chunk, 0)
            acc = lax.fori_loop(0, _CHUNKS, count_chunk, zeros)
            return (cur, jnp.sum(acc))

        lax.while_loop(cond, wbody, (jnp.int32(-1), jnp.int32(1)))
        pltpu.sync_copy(reach_v, out_hbm)


@jax.jit
def _reach(left, right):
    mesh = plsc.VectorSubcoreMesh(core_axis_name="c", subcore_axis_name="s")
    return pl.kernel(
        _reach_body,
        out_type=jax.ShapeDtypeStruct((_N,), jnp.int32),
        mesh=mesh,
        compiler_params=pltpu.CompilerParams(needs_layout_passes=False),
        scratch_types=[
            pltpu.VMEM((_N,), jnp.int32),
            pltpu.VMEM((_N,), jnp.int32),
            pltpu.VMEM((_N,), jnp.int32),
        ],
    )(left, right)


def kernel(thresholds, rules_left, rules_right, binary_ops, left, right,
           leaf_is_reroll, leaf_mask_left, leaf_mask_right, leaf_mask_op,
           leaf_score_cat):
    out = _reach(left.astype(jnp.int32), right.astype(jnp.int32))
    return out != 0
